# baseline (speedup 1.0000x reference)
import numpy as np
import jax
import jax.numpy as jnp
from jax.experimental import pallas as pl

B, NP, ND = 256, 55, 3
HID, TEMB, NLAYERS = 64, 64, 3
DATA_SIGMA = 0.5
COORDS_RANGE = 15.0
NPP = NP * NP
SP = 2
NLH = SP * HID
NLD = SP * ND
GRID = B // SP


def _selector_mats():
    i = np.repeat(np.arange(NP), NP)
    j = np.tile(np.arange(NP), NP)
    r_full = np.zeros((NPP, NP), np.float32)
    r_full[np.arange(NPP), i] = 1.0
    c_full = np.zeros((NPP, NP), np.float32)
    c_full[np.arange(NPP), j] = 1.0
    d_mat = r_full - c_full
    r0 = r_full.copy()
    r0[i == j] = 0.0
    return r0, c_full, d_mat


def _mm(a, b):
    return jax.lax.dot_general(a, b, (((1,), (0,)), ((), ())),
                               preferred_element_type=jnp.float32)


def _mmx(a, b):
    return jax.lax.dot_general(a, b, (((1,), (0,)), ((), ())),
                               precision=jax.lax.Precision.HIGHEST,
                               preferred_element_type=jnp.float32)


def _silu(v):
    return v * jax.nn.sigmoid(v)


def _kron2(w):
    eye = jnp.eye(SP, dtype=w.dtype)
    return jnp.einsum('se,...kn->...sken', eye, w).reshape(
        w.shape[:-2] + (SP * w.shape[-2], SP * w.shape[-1]))


def _egnn_body(lt6_ref, lt128_ref, xs_ref,
               gc_ref, d_ref, r0t_ref,
               w_emb_ref, b_emb_ref,
               w1ab_ref, pw_ref, b1_ref, w2_ref, b2_ref,
               attf_ref, attb_ref,
               nw1h_ref, nw1a_ref, nb1_ref, nw2_ref, nb2_ref,
               cw1_ref, cb1_ref, ce_ref,
               out_ref):
    lt6 = lt6_ref[0]
    t6 = jnp.exp(lt6)
    denom = DATA_SIGMA * DATA_SIGMA + t6 * t6
    c_in = 1.0 / jnp.sqrt(denom)
    c_skip = (DATA_SIGMA * DATA_SIGMA) / denom
    c_out = DATA_SIGMA * t6 / jnp.sqrt(denom)

    lt128 = lt128_ref[0]
    half = TEMB // 2
    kidx = jax.lax.broadcasted_iota(jnp.int32, (1, NLH), 1)
    k64 = kidx % HID
    kf = (k64 % half).astype(jnp.float32)
    freqs = jnp.exp(kf * np.float32(np.log(1.0 / 10000.0) / half))
    ang = (lt128 / 4.0) * freqs
    temb = jnp.where(k64 < half, jnp.cos(ang), jnp.sin(ang))
    h0 = _mm(temb, w_emb_ref[...]) + b_emb_ref[...]
    h = jnp.broadcast_to(h0, (NP, NLH))

    x_in = xs_ref[0]
    x0 = x_in * c_in
    x = x0

    gc = gc_ref[...]
    d_sel = d_ref[...]
    r0t = r0t_ref[...]

    rad_rep = ce_ref[NLAYERS, 0:NLD, :]
    d0 = _mmx(d_sel, x0)
    e6 = _mmx(d0 * d0, rad_rep)

    for i in range(NLAYERS):
        diff = _mmx(d_sel, x)
        rad6 = _mmx(diff * diff, rad_rep)
        cdiff = diff / (jnp.sqrt(rad6 + 1e-8) + 1.0)

        re12 = jnp.concatenate([rad6, e6], axis=1)

        a_rows = _mm(h, w1ab_ref[i, 0])
        b_rows = _mm(h, w1ab_ref[i, 1])
        ab = jnp.concatenate([a_rows, b_rows], axis=0)
        pre1 = _mmx(gc, ab) + _mm(re12, pw_ref[i]) + b1_ref[i]
        m1 = _silu(pre1)
        m2 = _silu(_mm(m1, w2_ref[i]) + b2_ref[i])
        att = jax.nn.sigmoid(_mm(m2, attf_ref[i]) + attb_ref[i])
        m = m2 * att

        cm = _silu(_mm(m, cw1_ref[i]) + cb1_ref[i])
        cw = jnp.tanh(_mm(cm, ce_ref[i]))
        trans = cdiff * (cw * COORDS_RANGE)

        x = x + _mmx(r0t, trans)
        agg = _mmx(r0t, m)

        n1 = _silu(_mm(h, nw1h_ref[i]) + _mm(agg, nw1a_ref[i]) + nb1_ref[i])
        h = h + _mm(n1, nw2_ref[i]) + nb2_ref[i]

    vel = x - x0
    out_ref[0] = x_in * c_skip + vel * c_out


def kernel(logt, xs, W_emb, b_emb, edge_w1, edge_b1, edge_w2, edge_b2,
           att_w, att_b, node_w1, node_b1, node_w2, node_b2,
           coord_w1, coord_b1, coord_w2):
    r0_np, c_np, d_np = _selector_mats()
    gc = jnp.asarray(np.concatenate([r0_np, c_np], axis=1))
    d_sel = jnp.asarray(d_np)
    r0t = jnp.asarray(np.ascontiguousarray(r0_np.T))

    lt6 = jnp.broadcast_to(logt.reshape(GRID, 1, SP, 1),
                           (GRID, 1, SP, ND)).reshape(GRID, 1, NLD)
    lt128 = jnp.broadcast_to(logt.reshape(GRID, 1, SP, 1),
                             (GRID, 1, SP, HID)).reshape(GRID, 1, NLH)
    xs6 = xs.reshape(GRID, SP, NP, ND).transpose(0, 2, 1, 3).reshape(
        GRID, NP, NLD)

    bd_w_emb = _kron2(W_emb)
    w1a = edge_w1[:, :HID, :]
    w1b = edge_w1[:, HID:2 * HID, :]
    w1ab = jnp.stack([_kron2(w1a), _kron2(w1b)], axis=1)
    w_r = edge_w1[:, 2 * HID, :]
    w_e = edge_w1[:, 2 * HID + 1, :]
    zpad = jnp.zeros((NLAYERS, ND - 1, HID), jnp.float32)
    pw_q = _kron2(jnp.concatenate([w_r[:, None, :], zpad], axis=1))
    pw_e = _kron2(jnp.concatenate([w_e[:, None, :], zpad], axis=1))
    pw = jnp.concatenate([pw_q, pw_e], axis=1)
    bd_w2 = _kron2(edge_w2)
    bd_cw1 = _kron2(coord_w1)
    bd_nw1h = _kron2(node_w1[:, :HID, :])
    bd_nw1a = _kron2(node_w1[:, HID:, :])
    bd_nw2 = _kron2(node_w2)
    attf = _kron2(att_w[:, :, 0:1] * jnp.ones((NLAYERS, 1, HID)))
    ce_l = _kron2(coord_w2 * jnp.ones((NLAYERS, 1, ND)))
    rad_rep = _kron2(jnp.ones((ND, ND), jnp.float32))[None]
    ce = jnp.concatenate(
        [ce_l, jnp.zeros((1,) + ce_l.shape[1:], jnp.float32).at[
            0, :NLD, :].set(rad_rep[0])], axis=0)

    def rep2(v):
        return jnp.concatenate([v, v], axis=1)[:, None, :]

    b1 = rep2(edge_b1)
    b2 = rep2(edge_b2)
    attb = jnp.broadcast_to(att_b[:, None, :], (NLAYERS, 1, NLH))
    nb1 = rep2(node_b1)
    nb2 = rep2(node_b2)
    cb1 = rep2(coord_b1)
    b_emb2 = jnp.concatenate([b_emb, b_emb])[None, :]

    grid = (GRID,)
    full = lambda *shape: pl.BlockSpec(
        shape, (lambda s: (0,) * len(shape)))
    in_specs = [
        pl.BlockSpec((1, 1, NLD), lambda s: (s, 0, 0)),
        pl.BlockSpec((1, 1, NLH), lambda s: (s, 0, 0)),
        pl.BlockSpec((1, NP, NLD), lambda s: (s, 0, 0)),
        full(NPP, 2 * NP),
        full(NPP, NP),
        full(NP, NPP),
        full(NLH, NLH),
        full(1, NLH),
        full(NLAYERS, 2, NLH, NLH),
        full(NLAYERS, 2 * NLD, NLH),
        full(NLAYERS, 1, NLH),
        full(NLAYERS, NLH, NLH),
        full(NLAYERS, 1, NLH),
        full(NLAYERS, NLH, NLH),
        full(NLAYERS, 1, NLH),
        full(NLAYERS, NLH, NLH),
        full(NLAYERS, NLH, NLH),
        full(NLAYERS, 1, NLH),
        full(NLAYERS, NLH, NLH),
        full(NLAYERS, 1, NLH),
        full(NLAYERS, NLH, NLH),
        full(NLAYERS, 1, NLH),
        full(NLAYERS + 1, NLH, NLD),
    ]
    out6 = pl.pallas_call(
        _egnn_body,
        grid=grid,
        in_specs=in_specs,
        out_specs=pl.BlockSpec((1, NP, NLD), lambda s: (s, 0, 0)),
        out_shape=jax.ShapeDtypeStruct((GRID, NP, NLD), jnp.float32),
    )(lt6, lt128, xs6, gc, d_sel, r0t, bd_w_emb, b_emb2,
      w1ab, pw, b1, bd_w2, b2, attf, attb,
      bd_nw1h, bd_nw1a, nb1, bd_nw2, nb2, bd_cw1, cb1, ce)
    return out6.reshape(GRID, NP, SP, ND).transpose(0, 2, 1, 3).reshape(
        B, NP * ND)


# speedup vs baseline: 4.2913x; 4.2913x over previous
import numpy as np
import jax
import jax.numpy as jnp
from jax.experimental import pallas as pl

B, NP, ND = 256, 55, 3
HID, TEMB, NLAYERS = 64, 64, 3
DATA_SIGMA = 0.5
COORDS_RANGE = 15.0
NPP = NP * NP
SP = 2
NLH = SP * HID
NLD = SP * ND
GRID = B // SP


def _selector_mats():
    i = np.repeat(np.arange(NP), NP)
    j = np.tile(np.arange(NP), NP)
    r_full = np.zeros((NPP, NP), np.float32)
    r_full[np.arange(NPP), i] = 1.0
    c_full = np.zeros((NPP, NP), np.float32)
    c_full[np.arange(NPP), j] = 1.0
    d_mat = r_full - c_full
    r0 = r_full.copy()
    r0[i == j] = 0.0
    return r0, c_full, d_mat


def _mm(a, b):
    return jax.lax.dot_general(a, b, (((1,), (0,)), ((), ())),
                               preferred_element_type=jnp.float32)


def _split2(v):
    hi = v.astype(jnp.bfloat16).astype(jnp.float32)
    return hi, v - hi


def _silu(v):
    return v * jax.nn.sigmoid(v)


def _kron2(w):
    eye = jnp.eye(SP, dtype=w.dtype)
    return jnp.einsum('se,...kn->...sken', eye, w).reshape(
        w.shape[:-2] + (SP * w.shape[-2], SP * w.shape[-1]))


def _egnn_body(lt6_ref, lt128_ref, xs_ref,
               gc2_ref, d2_ref, r0t_ref, rr_ref,
               w_emb_ref, b_emb_ref,
               w1ab_ref, pw_ref, b1_ref, w2_ref, b2_ref,
               attf_ref, attb_ref,
               nw1h_ref, nw1a_ref, nb1_ref, nw2_ref, nb2_ref,
               cw1_ref, cb1_ref, ce_ref,
               out_ref):
    lt6 = lt6_ref[0]
    t6 = jnp.exp(lt6)
    denom = DATA_SIGMA * DATA_SIGMA + t6 * t6
    c_in = 1.0 / jnp.sqrt(denom)
    c_skip = (DATA_SIGMA * DATA_SIGMA) / denom
    c_out = DATA_SIGMA * t6 / jnp.sqrt(denom)

    lt128 = lt128_ref[0]
    half = TEMB // 2
    kidx = jax.lax.broadcasted_iota(jnp.int32, (1, NLH), 1)
    k64 = kidx % HID
    kf = (k64 % half).astype(jnp.float32)
    freqs = jnp.exp(kf * np.float32(np.log(1.0 / 10000.0) / half))
    ang = (lt128 / 4.0) * freqs
    temb = jnp.where(k64 < half, jnp.cos(ang), jnp.sin(ang))
    h0 = _mm(temb, w_emb_ref[...]) + b_emb_ref[...]
    h = jnp.broadcast_to(h0, (NP, NLH))

    x_in = xs_ref[0]
    x0 = x_in * c_in
    x = x0

    gc2 = gc2_ref[...]
    d2 = d2_ref[...]
    r0t = r0t_ref[...]
    rr = rr_ref[...]

    def pair_diff(xc):
        hi, lo = _split2(xc)
        return _mm(d2, jnp.concatenate([hi, lo], axis=0))

    def rad_reduce(df):
        q = df * df
        hi, lo = _split2(q)
        return _mm(jnp.concatenate([hi, lo], axis=1), rr)

    def scatter(v):
        hi, lo = _split2(v)
        return _mm(r0t, hi) + _mm(r0t, lo)

    d0 = pair_diff(x0)
    e6 = rad_reduce(d0)

    for i in range(NLAYERS):
        diff = pair_diff(x)
        rad6 = rad_reduce(diff)
        cdiff = diff / (jnp.sqrt(rad6 + 1e-8) + 1.0)

        re12 = jnp.concatenate([rad6, e6], axis=1)

        a_rows = _mm(h, w1ab_ref[i, 0])
        b_rows = _mm(h, w1ab_ref[i, 1])
        ahi, alo = _split2(a_rows)
        bhi, blo = _split2(b_rows)
        ab = jnp.concatenate([ahi, bhi, alo, blo], axis=0)
        pre1 = _mm(gc2, ab) + _mm(re12, pw_ref[i]) + b1_ref[i]
        m1 = _silu(pre1)
        m2 = _silu(_mm(m1, w2_ref[i]) + b2_ref[i])
        att = jax.nn.sigmoid(_mm(m2, attf_ref[i]) + attb_ref[i])
        m = m2 * att

        cm = _silu(_mm(m, cw1_ref[i]) + cb1_ref[i])
        cw = jnp.tanh(_mm(cm, ce_ref[i]))
        trans = cdiff * (cw * COORDS_RANGE)

        x = x + scatter(trans)
        agg = scatter(m)

        n1 = _silu(_mm(h, nw1h_ref[i]) + _mm(agg, nw1a_ref[i]) + nb1_ref[i])
        h = h + _mm(n1, nw2_ref[i]) + nb2_ref[i]

    vel = x - x0
    out_ref[0] = x_in * c_skip + vel * c_out


def kernel(logt, xs, W_emb, b_emb, edge_w1, edge_b1, edge_w2, edge_b2,
           att_w, att_b, node_w1, node_b1, node_w2, node_b2,
           coord_w1, coord_b1, coord_w2):
    r0_np, c_np, d_np = _selector_mats()
    gc2 = jnp.asarray(np.concatenate([r0_np, c_np, r0_np, c_np], axis=1))
    d2 = jnp.asarray(np.concatenate([d_np, d_np], axis=1))
    r0t = jnp.asarray(np.ascontiguousarray(r0_np.T))
    rr_np = np.kron(np.eye(SP, dtype=np.float32),
                    np.ones((ND, ND), np.float32))
    rr = jnp.asarray(np.concatenate([rr_np, rr_np], axis=0))

    lt6 = jnp.broadcast_to(logt.reshape(GRID, 1, SP, 1),
                           (GRID, 1, SP, ND)).reshape(GRID, 1, NLD)
    lt128 = jnp.broadcast_to(logt.reshape(GRID, 1, SP, 1),
                             (GRID, 1, SP, HID)).reshape(GRID, 1, NLH)
    xs6 = xs.reshape(GRID, SP, NP, ND).transpose(0, 2, 1, 3).reshape(
        GRID, NP, NLD)

    bd_w_emb = _kron2(W_emb)
    w1a = edge_w1[:, :HID, :]
    w1b = edge_w1[:, HID:2 * HID, :]
    w1ab = jnp.stack([_kron2(w1a), _kron2(w1b)], axis=1)
    w_r = edge_w1[:, 2 * HID, :]
    w_e = edge_w1[:, 2 * HID + 1, :]
    zpad = jnp.zeros((NLAYERS, ND - 1, HID), jnp.float32)
    pw_q = _kron2(jnp.concatenate([w_r[:, None, :], zpad], axis=1))
    pw_e = _kron2(jnp.concatenate([w_e[:, None, :], zpad], axis=1))
    pw = jnp.concatenate([pw_q, pw_e], axis=1)
    bd_w2 = _kron2(edge_w2)
    bd_cw1 = _kron2(coord_w1)
    bd_nw1h = _kron2(node_w1[:, :HID, :])
    bd_nw1a = _kron2(node_w1[:, HID:, :])
    bd_nw2 = _kron2(node_w2)
    attf = _kron2(att_w[:, :, 0:1] * jnp.ones((NLAYERS, 1, HID)))
    ce = _kron2(coord_w2 * jnp.ones((NLAYERS, 1, ND)))

    def rep2(v):
        return jnp.concatenate([v, v], axis=1)[:, None, :]

    b1 = rep2(edge_b1)
    b2 = rep2(edge_b2)
    attb = jnp.broadcast_to(att_b[:, None, :], (NLAYERS, 1, NLH))
    nb1 = rep2(node_b1)
    nb2 = rep2(node_b2)
    cb1 = rep2(coord_b1)
    b_emb2 = jnp.concatenate([b_emb, b_emb])[None, :]

    grid = (GRID,)
    full = lambda *shape: pl.BlockSpec(
        shape, (lambda s: (0,) * len(shape)))
    in_specs = [
        pl.BlockSpec((1, 1, NLD), lambda s: (s, 0, 0)),
        pl.BlockSpec((1, 1, NLH), lambda s: (s, 0, 0)),
        pl.BlockSpec((1, NP, NLD), lambda s: (s, 0, 0)),
        full(NPP, 4 * NP),
        full(NPP, 2 * NP),
        full(NP, NPP),
        full(2 * NLD, NLD),
        full(NLH, NLH),
        full(1, NLH),
        full(NLAYERS, 2, NLH, NLH),
        full(NLAYERS, 2 * NLD, NLH),
        full(NLAYERS, 1, NLH),
        full(NLAYERS, NLH, NLH),
        full(NLAYERS, 1, NLH),
        full(NLAYERS, NLH, NLH),
        full(NLAYERS, 1, NLH),
        full(NLAYERS, NLH, NLH),
        full(NLAYERS, NLH, NLH),
        full(NLAYERS, 1, NLH),
        full(NLAYERS, NLH, NLH),
        full(NLAYERS, 1, NLH),
        full(NLAYERS, NLH, NLH),
        full(NLAYERS, 1, NLH),
        full(NLAYERS, NLH, NLD),
    ]
    out6 = pl.pallas_call(
        _egnn_body,
        grid=grid,
        in_specs=in_specs,
        out_specs=pl.BlockSpec((1, NP, NLD), lambda s: (s, 0, 0)),
        out_shape=jax.ShapeDtypeStruct((GRID, NP, NLD), jnp.float32),
    )(lt6, lt128, xs6, gc2, d2, r0t, rr, bd_w_emb, b_emb2,
      w1ab, pw, b1, bd_w2, b2, attf, attb,
      bd_nw1h, bd_nw1a, nb1, bd_nw2, nb2, bd_cw1, cb1, ce)
    return out6.reshape(GRID, NP, SP, ND).transpose(0, 2, 1, 3).reshape(
        B, NP * ND)


# speedup vs baseline: 4.4159x; 1.0290x over previous
import numpy as np
import jax
import jax.numpy as jnp
from jax.experimental import pallas as pl

B, NP, ND = 256, 55, 3
HID, TEMB, NLAYERS = 64, 64, 3
DATA_SIGMA = 0.5
COORDS_RANGE = 15.0
NPP = NP * NP
SP = 2
NLH = SP * HID
NLD = SP * ND
PAIRS = 2
GRID = B // (SP * PAIRS)


def _selector_mats():
    i = np.repeat(np.arange(NP), NP)
    j = np.tile(np.arange(NP), NP)
    r_full = np.zeros((NPP, NP), np.float32)
    r_full[np.arange(NPP), i] = 1.0
    c_full = np.zeros((NPP, NP), np.float32)
    c_full[np.arange(NPP), j] = 1.0
    d_mat = r_full - c_full
    r0 = r_full.copy()
    r0[i == j] = 0.0
    return r0, c_full, d_mat


def _mm(a, b):
    return jax.lax.dot_general(a, b, (((1,), (0,)), ((), ())),
                               preferred_element_type=jnp.float32)


def _split2(v):
    hi = v.astype(jnp.bfloat16).astype(jnp.float32)
    return hi, v - hi


def _silu(v):
    return v * jax.nn.sigmoid(v)


def _kron2(w):
    eye = jnp.eye(SP, dtype=w.dtype)
    return jnp.einsum('se,...kn->...sken', eye, w).reshape(
        w.shape[:-2] + (SP * w.shape[-2], SP * w.shape[-1]))


def _egnn_body(lt6_ref, lt128_ref, xs_ref,
               gc2_ref, d2_ref, r0t_ref, rr_ref,
               w_emb_ref, b_emb_ref,
               w1ab_ref, pw_ref, b1_ref, w2_ref, b2_ref,
               attf_ref, attb_ref,
               nw1h_ref, nw1a_ref, nb1_ref, nw2_ref, nb2_ref,
               cw1_ref, cb1_ref, ce_ref,
               out_ref):
    gc2 = gc2_ref[...]
    d2 = d2_ref[...]
    r0t = r0t_ref[...]
    rr = rr_ref[...]

    def pair_diff(xc):
        hi, lo = _split2(xc)
        return _mm(d2, jnp.concatenate([hi, lo], axis=0))

    def rad_reduce(df):
        q = df * df
        hi, lo = _split2(q)
        return _mm(jnp.concatenate([hi, lo], axis=1), rr)

    def scatter(v):
        hi, lo = _split2(v)
        return _mm(r0t, hi) + _mm(r0t, lo)

    half = TEMB // 2
    kidx = jax.lax.broadcasted_iota(jnp.int32, (1, NLH), 1)
    k64 = kidx % HID
    kf = (k64 % half).astype(jnp.float32)
    freqs = jnp.exp(kf * np.float32(np.log(1.0 / 10000.0) / half))

    for p in range(PAIRS):
        lt6 = lt6_ref[0, p]
        t6 = jnp.exp(lt6)
        denom = DATA_SIGMA * DATA_SIGMA + t6 * t6
        c_in = 1.0 / jnp.sqrt(denom)
        c_skip = (DATA_SIGMA * DATA_SIGMA) / denom
        c_out = DATA_SIGMA * t6 / jnp.sqrt(denom)

        lt128 = lt128_ref[0, p]
        ang = (lt128 / 4.0) * freqs
        temb = jnp.where(k64 < half, jnp.cos(ang), jnp.sin(ang))
        h0 = _mm(temb, w_emb_ref[...]) + b_emb_ref[...]
        h = jnp.broadcast_to(h0, (NP, NLH))

        x_in = xs_ref[0, p]
        x0 = x_in * c_in
        x = x0

        d0 = pair_diff(x0)
        e6 = rad_reduce(d0)

        for i in range(NLAYERS):
            diff = pair_diff(x)
            rad6 = rad_reduce(diff)
            cdiff = diff / (jnp.sqrt(rad6 + 1e-8) + 1.0)

            re12 = jnp.concatenate([rad6, e6], axis=1)

            a_rows = _mm(h, w1ab_ref[i, 0])
            b_rows = _mm(h, w1ab_ref[i, 1])
            ahi, alo = _split2(a_rows)
            bhi, blo = _split2(b_rows)
            ab = jnp.concatenate([ahi, bhi, alo, blo], axis=0)
            pre1 = _mm(gc2, ab) + _mm(re12, pw_ref[i]) + b1_ref[i]
            m1 = _silu(pre1)
            m2 = _silu(_mm(m1, w2_ref[i]) + b2_ref[i])
            att = jax.nn.sigmoid(_mm(m2, attf_ref[i]) + attb_ref[i])
            m = m2 * att

            cm = _silu(_mm(m, cw1_ref[i]) + cb1_ref[i])
            cw = jnp.tanh(_mm(cm, ce_ref[i]))
            trans = cdiff * (cw * COORDS_RANGE)

            x = x + scatter(trans)
            agg = scatter(m)

            n1 = _silu(_mm(h, nw1h_ref[i]) + _mm(agg, nw1a_ref[i])
                       + nb1_ref[i])
            h = h + _mm(n1, nw2_ref[i]) + nb2_ref[i]

        vel = x - x0
        out_ref[0, p] = x_in * c_skip + vel * c_out


def kernel(logt, xs, W_emb, b_emb, edge_w1, edge_b1, edge_w2, edge_b2,
           att_w, att_b, node_w1, node_b1, node_w2, node_b2,
           coord_w1, coord_b1, coord_w2):
    r0_np, c_np, d_np = _selector_mats()
    gc2 = jnp.asarray(np.concatenate([r0_np, c_np, r0_np, c_np], axis=1))
    d2 = jnp.asarray(np.concatenate([d_np, d_np], axis=1))
    r0t = jnp.asarray(np.ascontiguousarray(r0_np.T))
    rr_np = np.kron(np.eye(SP, dtype=np.float32),
                    np.ones((ND, ND), np.float32))
    rr = jnp.asarray(np.concatenate([rr_np, rr_np], axis=0))

    lt6 = jnp.broadcast_to(logt.reshape(GRID, PAIRS, 1, SP, 1),
                           (GRID, PAIRS, 1, SP, ND)).reshape(
        GRID, PAIRS, 1, NLD)
    lt128 = jnp.broadcast_to(logt.reshape(GRID, PAIRS, 1, SP, 1),
                             (GRID, PAIRS, 1, SP, HID)).reshape(
        GRID, PAIRS, 1, NLH)
    xs6 = xs.reshape(GRID, PAIRS, SP, NP, ND).transpose(
        0, 1, 3, 2, 4).reshape(GRID, PAIRS, NP, NLD)

    bd_w_emb = _kron2(W_emb)
    w1a = edge_w1[:, :HID, :]
    w1b = edge_w1[:, HID:2 * HID, :]
    w1ab = jnp.stack([_kron2(w1a), _kron2(w1b)], axis=1)
    w_r = edge_w1[:, 2 * HID, :]
    w_e = edge_w1[:, 2 * HID + 1, :]
    zpad = jnp.zeros((NLAYERS, ND - 1, HID), jnp.float32)
    pw_q = _kron2(jnp.concatenate([w_r[:, None, :], zpad], axis=1))
    pw_e = _kron2(jnp.concatenate([w_e[:, None, :], zpad], axis=1))
    pw = jnp.concatenate([pw_q, pw_e], axis=1)
    bd_w2 = _kron2(edge_w2)
    bd_cw1 = _kron2(coord_w1)
    bd_nw1h = _kron2(node_w1[:, :HID, :])
    bd_nw1a = _kron2(node_w1[:, HID:, :])
    bd_nw2 = _kron2(node_w2)
    attf = _kron2(att_w[:, :, 0:1] * jnp.ones((NLAYERS, 1, HID)))
    ce = _kron2(coord_w2 * jnp.ones((NLAYERS, 1, ND)))

    def rep2(v):
        return jnp.concatenate([v, v], axis=1)[:, None, :]

    b1 = rep2(edge_b1)
    b2 = rep2(edge_b2)
    attb = jnp.broadcast_to(att_b[:, None, :], (NLAYERS, 1, NLH))
    nb1 = rep2(node_b1)
    nb2 = rep2(node_b2)
    cb1 = rep2(coord_b1)
    b_emb2 = jnp.concatenate([b_emb, b_emb])[None, :]

    grid = (GRID,)
    full = lambda *shape: pl.BlockSpec(
        shape, (lambda s: (0,) * len(shape)))
    in_specs = [
        pl.BlockSpec((1, PAIRS, 1, NLD), lambda s: (s, 0, 0, 0)),
        pl.BlockSpec((1, PAIRS, 1, NLH), lambda s: (s, 0, 0, 0)),
        pl.BlockSpec((1, PAIRS, NP, NLD), lambda s: (s, 0, 0, 0)),
        full(NPP, 4 * NP),
        full(NPP, 2 * NP),
        full(NP, NPP),
        full(2 * NLD, NLD),
        full(NLH, NLH),
        full(1, NLH),
        full(NLAYERS, 2, NLH, NLH),
        full(NLAYERS, 2 * NLD, NLH),
        full(NLAYERS, 1, NLH),
        full(NLAYERS, NLH, NLH),
        full(NLAYERS, 1, NLH),
        full(NLAYERS, NLH, NLH),
        full(NLAYERS, 1, NLH),
        full(NLAYERS, NLH, NLH),
        full(NLAYERS, NLH, NLH),
        full(NLAYERS, 1, NLH),
        full(NLAYERS, NLH, NLH),
        full(NLAYERS, 1, NLH),
        full(NLAYERS, NLH, NLH),
        full(NLAYERS, 1, NLH),
        full(NLAYERS, NLH, NLD),
    ]
    out6 = pl.pallas_call(
        _egnn_body,
        grid=grid,
        in_specs=in_specs,
        out_specs=pl.BlockSpec((1, PAIRS, NP, NLD),
                               lambda s: (s, 0, 0, 0)),
        out_shape=jax.ShapeDtypeStruct((GRID, PAIRS, NP, NLD), jnp.float32),
    )(lt6, lt128, xs6, gc2, d2, r0t, rr, bd_w_emb, b_emb2,
      w1ab, pw, b1, bd_w2, b2, attf, attb,
      bd_nw1h, bd_nw1a, nb1, bd_nw2, nb2, bd_cw1, cb1, ce)
    return out6.reshape(GRID, PAIRS, NP, SP, ND).transpose(
        0, 1, 3, 2, 4).reshape(B, NP * ND)
